# baseline (speedup 1.0000x reference)
import functools

import numpy as np

import jax
import jax.numpy as jnp
from jax import lax
from jax.experimental import pallas as pl
from jax.experimental.pallas import tpu as pltpu
from jax.experimental.pallas import tpu_sc as plsc

_D = 2048
_HD = (13, 128, 128)
_B = 64
_PAD = 128
_NEG = -1e9
_NROW = 2
_NT = 32
_LN2 = 0.6931471805599453


def _tf2x32(k1, k2, c1, c2):
    rot = [np.uint32(r) for r in (13, 15, 26, 6, 17, 29, 16, 24)]

    def rotl(x, d):
        return (x << d) | (x >> np.uint32(32 - d))

    ks0, ks1 = np.uint32(k1), np.uint32(k2)
    ks2 = ks0 ^ ks1 ^ np.uint32(0x1BD11BDA)
    x0 = (c1 + ks0).astype(np.uint32)
    x1 = (c2 + ks1).astype(np.uint32)
    ks = [ks0, ks1, ks2]
    rsets = [rot[0:4], rot[4:8]]
    with np.errstate(over="ignore"):
        for i in range(5):
            for r in rsets[i % 2]:
                x0 = (x0 + x1).astype(np.uint32)
                x1 = rotl(x1, r)
                x1 = x1 ^ x0
            x0 = (x0 + ks[(i + 1) % 3]).astype(np.uint32)
            x1 = (x1 + ks[(i + 2) % 3] + np.uint32(i + 1)).astype(np.uint32)
    return x0, x1


def _gumbel_np(head):
    k = _tf2x32(0, 42, np.uint32([0]), np.uint32([head]))
    size = _B * _HD[head]
    idx = np.arange(size, dtype=np.uint64)
    c1 = (idx >> np.uint64(32)).astype(np.uint32)
    c2 = (idx & np.uint64(0xFFFFFFFF)).astype(np.uint32)
    b1, b2 = _tf2x32(k[0][0], k[1][0], c1, c2)
    f = (((b1 ^ b2) >> np.uint32(9)) | np.uint32(0x3F800000)).view(np.float32)
    f = f - np.float32(1.0)
    tiny = np.float32(np.finfo(np.float32).tiny)
    u = np.maximum(tiny, f * (np.float32(1.0) - tiny) + tiny)
    return (-np.log(-np.log(u))).reshape(_B, _HD[head]).astype(np.float32)


def _gcat_np():
    g = np.zeros((_B, 3 * _PAD), np.float32)
    g[:, 0:_HD[0]] = _gumbel_np(0)
    g[:, _PAD:2 * _PAD] = _gumbel_np(1)
    g[:, 2 * _PAD:3 * _PAD] = _gumbel_np(2)
    return g


_GCAT = _gcat_np()


def _tc_body(x_ref, m0t_ref, m1_ref, m2_ref, w0t_ref, b0_ref, w1_ref, b1_ref,
             w2_ref, b2_ref, gcat_ref, lm_ref, z_ref, t01_ref, w2b1_ref):
    x = x_ref[:]
    y0 = (lax.dot_general(x, w0t_ref[:], (((1,), (1,)), ((), ())),
                          preferred_element_type=jnp.float32)
          + b0_ref[:][None, :])
    lm0 = jnp.where(m0t_ref[:].T > 0, y0, _NEG)
    lm0 = jnp.pad(lm0, ((0, 0), (0, _PAD - _HD[0])), constant_values=_NEG)
    y1 = (jnp.dot(x, w1_ref[pl.ds(0, _D), :], preferred_element_type=jnp.float32)
          + b1_ref[:][None, :])
    lm1 = jnp.where(m1_ref[:] > 0, y1, _NEG)
    y2 = (jnp.dot(x, w2_ref[pl.ds(0, _D), :], preferred_element_type=jnp.float32)
          + b2_ref[:][None, :])
    lm2 = jnp.where(m2_ref[:] > 0, y2, _NEG)
    lm = jnp.concatenate([lm0, lm1, lm2], axis=1)
    lm_ref[:] = lm
    z_ref[:] = lm + gcat_ref[:]
    t01_ref[:] = jnp.pad(
        jnp.concatenate([w1_ref[pl.ds(_D, _HD[0]), :],
                         w2_ref[pl.ds(_D, _HD[0]), :]], axis=1),
        ((0, 16 - _HD[0]), (0, 0)))
    w2b1_ref[:] = w2_ref[pl.ds(_D + _HD[0], _HD[1]), :]


def _recip_f32(x):
    r = lax.bitcast_convert_type(0x7EF311C3 - lax.bitcast_convert_type(x, jnp.int32),
                                 jnp.float32)
    for _ in range(3):
        r = r * (2.0 - x * r)
    return r


def _log_f32(s):
    bits = lax.bitcast_convert_type(s, jnp.int32)
    e = lax.shift_right_logical(bits, 23) - 127
    mbits = lax.bitwise_or(lax.bitwise_and(bits, 0x007FFFFF), 0x3F800000)
    m = lax.bitcast_convert_type(mbits, jnp.float32)
    big = m > 1.4142135
    m = jnp.where(big, m * 0.5, m)
    e = jnp.where(big, e + 1, e)
    t = (m - 1.0) * _recip_f32(m + 1.0)
    t2 = t * t
    p = 2.0 * t * (1.0 + t2 * (0.33333333 + t2 * (0.2 + t2 * (0.14285714
                                                              + t2 * 0.11111111))))
    return e.astype(jnp.float32) * _LN2 + p


def _sample_head(read_lm, read_z, extra):
    iota = lax.iota(jnp.int32, 16)
    lms, zs = [], []
    for j in range(8):
        lmj = read_lm(j)
        zj = read_z(j)
        if extra is not None:
            ex = extra(j)
            lmj = lmj + ex
            zj = zj + ex
        lms.append(lmj)
        zs.append(zj)
    zacc, macc = zs[0], lms[0]
    for j in range(1, 8):
        zacc = jnp.maximum(zacc, zs[j])
        macc = jnp.maximum(macc, lms[j])
    zmax = jnp.max(zacc)
    mx = jnp.max(macc)
    idxacc = jnp.full((16,), 1 << 20, jnp.int32)
    sacc = jnp.zeros((16,), jnp.float32)
    aacc = jnp.zeros((16,), jnp.float32)
    for j in range(8):
        cand = iota + (16 * j)
        idxacc = jnp.minimum(idxacc, jnp.where(zs[j] >= zmax, cand, 1 << 20))
        sh = lms[j] - mx
        ej = jnp.exp(sh)
        sacc = sacc + ej
        aacc = aacc + ej * sh
    a = jnp.min(idxacc)
    lpacc = jnp.zeros((16,), jnp.float32)
    for j in range(8):
        cand = iota + (16 * j)
        lpacc = lpacc + jnp.where(cand == a, lms[j] - mx, 0.0)
    s = jnp.sum(sacc)
    logs = _log_f32(s)
    lp = jnp.sum(lpacc) - logs
    ent = logs - jnp.sum(aacc) * _recip_f32(s)
    return a, lp, ent


def _sc_body(lm_hbm, z_hbm, t01_hbm, w2b1_hbm, out_hbm,
             lm_v, z_v, t01_v, w2b1_v, out_v, sem):
    wid = lax.axis_index("s") * 2 + lax.axis_index("c")
    pltpu.sync_copy(lm_hbm.at[pl.ds(wid * _NROW, _NROW)], lm_v)
    pltpu.sync_copy(z_hbm.at[pl.ds(wid * _NROW, _NROW)], z_v)
    iota = lax.iota(jnp.int32, 16)

    def rd(ref, r, col0):
        return lambda j: ref[r, pl.ds(col0 + 16 * j, 16)]

    a0, res0 = [], []
    for r in range(_NROW):
        a, lp, ent = _sample_head(rd(lm_v, r, 0), rd(z_v, r, 0), None)
        a0.append(a)
        res0.append((lp, ent))
    avec = jnp.where(iota == 0, a0[0], jnp.where(iota == 1, a0[1], 0))
    pltpu.async_copy(t01_hbm.at[avec], t01_v, sem).wait()

    a1, res1 = [], []
    for r in range(_NROW):
        ex = lambda j, r=r: t01_v[r, pl.ds(16 * j, 16)]
        a, lp, ent = _sample_head(rd(lm_v, r, 128), rd(z_v, r, 128), ex)
        a1.append(a)
        res1.append((lp, ent))
    avec1 = jnp.where(iota == 0, a1[0], jnp.where(iota == 1, a1[1], 0))
    pltpu.async_copy(w2b1_hbm.at[avec1], w2b1_v, sem).wait()

    res2 = []
    for r in range(_NROW):
        ex = lambda j, r=r: (t01_v[r, pl.ds(128 + 16 * j, 16)]
                             + w2b1_v[r, pl.ds(16 * j, 16)])
        _, lp, ent = _sample_head(rd(lm_v, r, 256), rd(z_v, r, 256), ex)
        res2.append((lp, ent))

    lp_r0 = res0[0][0] + res1[0][0] + res2[0][0]
    en_r0 = res0[0][1] + res1[0][1] + res2[0][1]
    lp_r1 = res0[1][0] + res1[1][0] + res2[1][0]
    en_r1 = res0[1][1] + res1[1][1] + res2[1][1]
    ovec = (jnp.where(iota == 0, lp_r0, 0.0)
            + jnp.where(iota == 1, en_r0, 0.0)
            + jnp.where(iota == 2, lp_r1, 0.0)
            + jnp.where(iota == 3, en_r1, 0.0))
    out_v[...] = ovec
    pltpu.sync_copy(out_v, out_hbm.at[wid])


def kernel(main_input, mask0, mask1, mask2, W0, b0, W1, b1, W2, b2):
    lm, z, t01, w2b1 = pl.pallas_call(
        _tc_body,
        out_shape=(jax.ShapeDtypeStruct((_B, 3 * _PAD), jnp.float32),
                   jax.ShapeDtypeStruct((_B, 3 * _PAD), jnp.float32),
                   jax.ShapeDtypeStruct((16, 2 * _PAD), jnp.float32),
                   jax.ShapeDtypeStruct((_PAD, _PAD), jnp.float32)),
    )(main_input, mask0.T, mask1, mask2,
      W0.T, b0, W1, b1, W2, b2, jnp.asarray(_GCAT))

    sc = functools.partial(
        pl.kernel,
        mesh=plsc.VectorSubcoreMesh(core_axis_name="c", subcore_axis_name="s"),
        compiler_params=pltpu.CompilerParams(needs_layout_passes=False),
        out_type=jax.ShapeDtypeStruct((_NT, 16), jnp.float32),
        scratch_types=[
            pltpu.VMEM((_NROW, 384), jnp.float32),
            pltpu.VMEM((_NROW, 384), jnp.float32),
            pltpu.VMEM((16, 256), jnp.float32),
            pltpu.VMEM((16, 128), jnp.float32),
            pltpu.VMEM((16,), jnp.float32),
            pltpu.SemaphoreType.DMA,
        ],
    )(_sc_body)
    out = sc(lm, z, t01, w2b1)
    return out[:, :4].reshape(_B, 2)


# speedup vs baseline: 8.8018x; 8.8018x over previous
import numpy as np

import jax
import jax.numpy as jnp
from jax import lax
from jax.experimental import pallas as pl

_D = 2048
_HD = (13, 128, 128)
_B = 64
_NEG = -1e9


def _tf2x32(k1, k2, c1, c2):
    rot = [np.uint32(r) for r in (13, 15, 26, 6, 17, 29, 16, 24)]

    def rotl(x, d):
        return (x << d) | (x >> np.uint32(32 - d))

    ks0, ks1 = np.uint32(k1), np.uint32(k2)
    ks2 = ks0 ^ ks1 ^ np.uint32(0x1BD11BDA)
    x0 = (c1 + ks0).astype(np.uint32)
    x1 = (c2 + ks1).astype(np.uint32)
    ks = [ks0, ks1, ks2]
    rsets = [rot[0:4], rot[4:8]]
    with np.errstate(over="ignore"):
        for i in range(5):
            for r in rsets[i % 2]:
                x0 = (x0 + x1).astype(np.uint32)
                x1 = rotl(x1, r)
                x1 = x1 ^ x0
            x0 = (x0 + ks[(i + 1) % 3]).astype(np.uint32)
            x1 = (x1 + ks[(i + 2) % 3] + np.uint32(i + 1)).astype(np.uint32)
    return x0, x1


def _gumbel_np(head):
    k = _tf2x32(0, 42, np.uint32([0]), np.uint32([head]))
    size = _B * _HD[head]
    idx = np.arange(size, dtype=np.uint64)
    c1 = (idx >> np.uint64(32)).astype(np.uint32)
    c2 = (idx & np.uint64(0xFFFFFFFF)).astype(np.uint32)
    b1, b2 = _tf2x32(k[0][0], k[1][0], c1, c2)
    f = (((b1 ^ b2) >> np.uint32(9)) | np.uint32(0x3F800000)).view(np.float32)
    f = f - np.float32(1.0)
    tiny = np.float32(np.finfo(np.float32).tiny)
    u = np.maximum(tiny, f * (np.float32(1.0) - tiny) + tiny)
    return (-np.log(-np.log(u))).reshape(_B, _HD[head]).astype(np.float32)


_G = tuple(_gumbel_np(i) for i in range(3))


def _head_stats(lm, g):
    d = lm.shape[1]
    col = jax.lax.broadcasted_iota(jnp.int32, lm.shape, 1)
    z = lm + g
    zmax = jnp.max(z, axis=1, keepdims=True)
    a = jnp.min(jnp.where(z >= zmax, col, d), axis=1, keepdims=True)
    onehot = (col == a).astype(jnp.float32)
    mx = jnp.max(lm, axis=1, keepdims=True)
    e = jnp.exp(lm - mx)
    s = jnp.sum(e, axis=1, keepdims=True)
    lse = mx + jnp.log(s)
    lp_vec = lm - lse
    lp = jnp.sum(onehot * lp_vec, axis=1, keepdims=True)
    ent = -jnp.sum((e / s) * lp_vec, axis=1, keepdims=True)
    return onehot, lp, ent


def _body(x_ref, w0t_ref, w1_ref, w2_ref, m0t_ref, m1_ref, m2_ref,
          b0_ref, b1_ref, b2_ref, g0_ref, g1_ref, g2_ref, out_ref):
    x = x_ref[:]

    y0 = (lax.dot_general(x, w0t_ref[:], (((1,), (1,)), ((), ())),
                          preferred_element_type=jnp.float32)
          + b0_ref[:][None, :])
    lm0 = jnp.where(m0t_ref[:].T > 0, y0, _NEG)
    oh0, lp0, ent0 = _head_stats(lm0, g0_ref[:])

    y1 = (jnp.dot(x, w1_ref[pl.ds(0, _D), :], preferred_element_type=jnp.float32)
          + jnp.dot(oh0, w1_ref[pl.ds(_D, _HD[0]), :],
                    preferred_element_type=jnp.float32)
          + b1_ref[:][None, :])
    lm1 = jnp.where(m1_ref[:] > 0, y1, _NEG)
    oh1, lp1, ent1 = _head_stats(lm1, g1_ref[:])

    ohx = jnp.concatenate([oh0, oh1], axis=1)
    y2 = (jnp.dot(x, w2_ref[pl.ds(0, _D), :], preferred_element_type=jnp.float32)
          + jnp.dot(ohx, w2_ref[pl.ds(_D, _HD[0] + _HD[1]), :],
                    preferred_element_type=jnp.float32)
          + b2_ref[:][None, :])
    lm2 = jnp.where(m2_ref[:] > 0, y2, _NEG)
    _, lp2, ent2 = _head_stats(lm2, g2_ref[:])

    lp = lp0 + lp1 + lp2
    ent = ent0 + ent1 + ent2
    col = jax.lax.broadcasted_iota(jnp.int32, (_B, 128), 1)
    out_ref[:] = jnp.where(col == 0, lp, jnp.where(col == 1, ent, 0.0))


def kernel(main_input, mask0, mask1, mask2, W0, b0, W1, b1, W2, b2):
    out = pl.pallas_call(
        _body,
        out_shape=jax.ShapeDtypeStruct((_B, 128), jnp.float32),
    )(main_input, W0.T, W1, W2, mask0.T, mask1, mask2, b0, b1, b2,
      jnp.asarray(_G[0]), jnp.asarray(_G[1]), jnp.asarray(_G[2]))
    return out[:, :2]
